# baseline (speedup 1.0000x reference)
import functools
import numpy as np
import jax
import jax.numpy as jnp
from jax import lax
from jax.experimental import pallas as pl
from jax.experimental.pallas import tpu as pltpu
from jax.experimental.pallas import tpu_sc as plsc

_VOCAB = 100000
_D = 768
_MAX_SEQ = 2048
_SCALE = float(np.sqrt(np.float32(_D)))

_NC = 2
_NS = 16
_NW = _NC * _NS
_LANES = 16

_CHUNK = 32


def _sinus_pe(max_len, d_model):
    pos = np.arange(max_len, dtype=np.float32)[:, None]
    div = np.exp(np.arange(0, d_model, 2, dtype=np.float32) * (-np.log(10000.0) / d_model))
    pe = np.zeros((max_len, d_model), dtype=np.float32)
    pe[:, 0::2] = np.sin(pos * div)
    pe[:, 1::2] = np.cos(pos * div)
    return pe


_PE = _sinus_pe(_MAX_SEQ, _D)


def _make_emb_kernel(batch, seq_len):
    n_rows = batch * seq_len
    assert seq_len % _NW == 0
    pos_per_w = seq_len // _NW
    c_p = 16
    assert pos_per_w % c_p == 0
    n_chunks = pos_per_w // c_p

    mesh = plsc.VectorSubcoreMesh(
        core_axis_name="c", subcore_axis_name="s",
        num_cores=_NC, num_subcores=_NS)

    @functools.partial(
        pl.kernel,
        out_type=jax.ShapeDtypeStruct((n_rows, _D), jnp.float32),
        mesh=mesh,
        scratch_types=[
            pltpu.VMEM((batch * pos_per_w,), jnp.int32),
            [pltpu.VMEM((batch * c_p, _D), jnp.float32) for _ in range(2)],
            [pltpu.VMEM((c_p, _D), jnp.float32) for _ in range(2)],
            [pltpu.SemaphoreType.DMA for _ in range(2)],
            [pltpu.SemaphoreType.DMA for _ in range(2)],
            [pltpu.SemaphoreType.DMA for _ in range(2)],
        ],
    )
    def emb(x_hbm, pe_hbm, table_hbm, out_hbm,
            idx_v, rows_v, pe_v, gsem, psem, osem):
        wid = lax.axis_index("s") * _NC + lax.axis_index("c")
        p_base = wid * pos_per_w
        for b in range(batch):
            pltpu.sync_copy(
                x_hbm.at[pl.ds(b * seq_len + p_base, pos_per_w)],
                idx_v.at[pl.ds(b * pos_per_w, pos_per_w)])

        def gather_copies(c):
            buf = c % 2
            return [
                pltpu.make_async_copy(
                    table_hbm.at[idx_v.at[pl.ds(b * pos_per_w + c * c_p, c_p)]],
                    rows_v[buf].at[pl.ds(b * c_p, c_p)],
                    gsem[buf])
                for b in range(batch)
            ]

        def pe_copy(c):
            buf = c % 2
            return pltpu.make_async_copy(
                pe_hbm.at[pl.ds(p_base + c * c_p, c_p)], pe_v[buf], psem[buf])

        def store_copies(c):
            buf = c % 2
            return [
                pltpu.make_async_copy(
                    rows_v[buf].at[pl.ds(b * c_p, c_p)],
                    out_hbm.at[pl.ds(b * seq_len + p_base + c * c_p, c_p)],
                    osem[buf])
                for b in range(batch)
            ]

        def start_chunk(c):
            for cp in gather_copies(c):
                cp.start()
            pe_copy(c).start()

        start_chunk(0)
        start_chunk(1)
        for c in range(n_chunks):
            buf = c % 2
            for cp in gather_copies(c):
                cp.wait()
            pe_copy(c).wait()

            def row_body(r, _, buf=buf):
                for j in range(_D // _LANES):
                    sl = pl.ds(j * _LANES, _LANES)
                    v = pe_v[buf][r, sl]
                    for b in range(batch):
                        rb = b * c_p + r
                        rows_v[buf][rb, sl] = rows_v[buf][rb, sl] * _SCALE + v
                return 0

            lax.fori_loop(0, c_p, row_body, 0)
            for cp in store_copies(c):
                cp.start()
            if c + 2 < n_chunks:
                for cp in store_copies(c):
                    cp.wait()
                start_chunk(c + 2)
        for c in (n_chunks - 2, n_chunks - 1):
            for cp in store_copies(c):
                cp.wait()

    return emb


@jax.jit
def kernel(x, table):
    batch, seq_len = x.shape
    x_flat = x.reshape(-1).astype(jnp.int32)
    pe = jnp.asarray(_PE[:seq_len])
    out = _make_emb_kernel(batch, seq_len)(x_flat, pe, table)
    return out.reshape(batch, seq_len, _D)


# speedup vs baseline: 1.3083x; 1.3083x over previous
import functools
import numpy as np
import jax
import jax.numpy as jnp
from jax import lax
from jax.experimental import pallas as pl
from jax.experimental.pallas import tpu as pltpu
from jax.experimental.pallas import tpu_sc as plsc

_VOCAB = 100000
_D = 768
_MAX_SEQ = 2048
_SCALE = float(np.sqrt(np.float32(_D)))

_NC = 2
_NS = 16
_NW = _NC * _NS
_LANES = 16

_CHUNK = 32


def _sinus_pe(max_len, d_model):
    pos = np.arange(max_len, dtype=np.float32)[:, None]
    div = np.exp(np.arange(0, d_model, 2, dtype=np.float32) * (-np.log(10000.0) / d_model))
    pe = np.zeros((max_len, d_model), dtype=np.float32)
    pe[:, 0::2] = np.sin(pos * div)
    pe[:, 1::2] = np.cos(pos * div)
    return pe


_PE = _sinus_pe(_MAX_SEQ, _D)


def _make_emb_kernel(batch, seq_len):
    n_rows = batch * seq_len
    assert n_rows % _NW == 0
    rows_per_w = n_rows // _NW
    assert rows_per_w % _CHUNK == 0
    n_chunks = rows_per_w // _CHUNK
    assert seq_len % rows_per_w == 0
    _NB = 3

    mesh = plsc.VectorSubcoreMesh(
        core_axis_name="c", subcore_axis_name="s",
        num_cores=_NC, num_subcores=_NS)

    @functools.partial(
        pl.kernel,
        out_type=jax.ShapeDtypeStruct((n_rows, _D), jnp.float32),
        mesh=mesh,
        scratch_types=[
            pltpu.VMEM((rows_per_w,), jnp.int32),
            [pltpu.VMEM((_CHUNK, _D), jnp.float32) for _ in range(_NB)],
            [pltpu.VMEM((_CHUNK, _D), jnp.float32) for _ in range(2)],
            [pltpu.SemaphoreType.DMA for _ in range(_NB)],
            [pltpu.SemaphoreType.DMA for _ in range(2)],
            [pltpu.SemaphoreType.DMA for _ in range(_NB)],
        ],
    )
    def emb(x_hbm, pe_hbm, table_hbm, out_hbm,
            idx_v, rows_v, pe_v, gsem, psem, osem):
        wid = lax.axis_index("s") * _NC + lax.axis_index("c")
        base = wid * rows_per_w
        s_base = lax.rem(base, seq_len)
        pltpu.sync_copy(x_hbm.at[pl.ds(base, rows_per_w)], idx_v)

        def gather_copy(c):
            buf = c % _NB
            return pltpu.make_async_copy(
                table_hbm.at[idx_v.at[pl.ds(c * _CHUNK, _CHUNK)]],
                rows_v[buf], gsem[buf])

        def pe_copy(c):
            buf = c % 2
            return pltpu.make_async_copy(
                pe_hbm.at[pl.ds(s_base + c * _CHUNK, _CHUNK)],
                pe_v[buf], psem[buf])

        def store_copy(c):
            buf = c % _NB
            return pltpu.make_async_copy(
                rows_v[buf], out_hbm.at[pl.ds(base + c * _CHUNK, _CHUNK)],
                osem[buf])

        gather_copy(0).start()
        pe_copy(0).start()
        gather_copy(1).start()
        pe_copy(1).start()
        for c in range(n_chunks):
            buf = c % _NB
            gather_copy(c).wait()
            pe_copy(c).wait()

            def row_body(r, _, buf=buf, pb=c % 2):
                for j in range(_D // _LANES):
                    sl = pl.ds(j * _LANES, _LANES)
                    rows_v[buf][r, sl] = rows_v[buf][r, sl] * _SCALE + pe_v[pb][r, sl]
                return 0

            lax.fori_loop(0, _CHUNK, row_body, 0)
            store_copy(c).start()
            if c + 2 < n_chunks:
                pe_copy(c + 2).start()
                if c >= 1:
                    store_copy(c - 1).wait()
                gather_copy(c + 2).start()
        store_copy(n_chunks - 3).wait()
        store_copy(n_chunks - 2).wait()
        store_copy(n_chunks - 1).wait()

    return emb


@jax.jit
def kernel(x, table):
    batch, seq_len = x.shape
    x_flat = x.reshape(-1).astype(jnp.int32)
    pe = jnp.asarray(_PE[:seq_len])
    out = _make_emb_kernel(batch, seq_len)(x_flat, pe, table)
    return out.reshape(batch, seq_len, _D)
